# baseline (speedup 1.0000x reference)
import jax
import jax.numpy as jnp
from jax.experimental import pallas as pl

_L = 200
_DIN = 64
_DATT = 128
_H = 4
_HD = 32
_K = 50
_BB = 128


def _limbs(a):
    a0 = a.astype(jnp.bfloat16)
    r1 = a - a0.astype(jnp.float32)
    a1 = r1.astype(jnp.bfloat16)
    r2 = r1 - a1.astype(jnp.float32)
    a2 = r2.astype(jnp.bfloat16)
    return a0, a1, a2


def _body(t_ref, x_ref, wq_ref, wh_ref, wv_ref, wo_ref, o_ref):
    tgt = t_ref[...]
    Xp = x_ref[...]
    Wq = wq_ref[...]
    Wh = wh_ref[...]
    Wv = wv_ref[...]
    Wo = wo_ref[...]
    bb = tgt.shape[0]

    Xe = Xp[:, :, :_DIN]
    Xo = Xp[:, :, _DIN:]
    qf = jax.lax.dot_general(tgt.astype(jnp.bfloat16),
                             Wq.astype(jnp.bfloat16),
                             (((1,), (1,)), ((), ())),
                             preferred_element_type=jnp.float32)
    Whbf = Wh.astype(jnp.bfloat16)
    rowh = jax.lax.broadcasted_iota(jnp.int32, (_DATT, _DIN), 0) // _HD
    Whbd = jnp.stack([jnp.where(rowh == h, Whbf, jnp.bfloat16(0))
                      for h in range(_H)], axis=1)
    qtil = jnp.zeros((bb, _H, _DIN), jnp.float32)
    for limb in _limbs(qf):
        qtil = qtil + jax.lax.dot_general(
            limb, Whbd, (((1,), (0,)), ((), ())),
            preferred_element_type=jnp.float32)
    se = jnp.zeros((bb, _H, _L // 2), jnp.float32)
    so = jnp.zeros((bb, _H, _L // 2), jnp.float32)
    for limb in _limbs(qtil):
        se = se + jax.lax.dot_general(
            limb, Xe, (((2,), (2,)), ((0,), (0,))),
            preferred_element_type=jnp.float32)
        so = so + jax.lax.dot_general(
            limb, Xo, (((2,), (2,)), ((0,), (0,))),
            preferred_element_type=jnp.float32)
    scores3 = jnp.concatenate([se, so], axis=-1)
    sT = jnp.swapaxes(scores3.reshape(bb * _H, _L), 0, 1) / (_HD ** 0.5)

    ibits = jax.lax.bitcast_convert_type(sT, jnp.int32)
    ikeys = jnp.where(ibits >= 0, ibits,
                      jnp.bitwise_xor(jnp.bitwise_not(ibits),
                                      jnp.int32(-2 ** 31)))

    t0 = jnp.full((1, bb * _H), jnp.int32(-2 ** 31), jnp.int32)

    def bs_body(i, t):
        inc = jax.lax.shift_left(jnp.int32(1), jnp.int32(31) - i)
        cand = t + inc
        cnt = jnp.sum((ikeys >= cand).astype(jnp.int32), axis=0,
                      keepdims=True)
        return jnp.where(cnt >= _K, cand, t)

    thr = jax.lax.fori_loop(0, 32, bs_body, t0)
    keep = ikeys >= thr

    m = jnp.max(sT, axis=0, keepdims=True)
    p = jnp.where(keep, jnp.exp(sT - m), 0.0)
    attnT = p / jnp.sum(p, axis=0, keepdims=True)
    attn = jnp.swapaxes(attnT.astype(jnp.bfloat16), 0, 1).reshape(bb, _H, _L)

    mixed = (jax.lax.dot_general(
        attn[:, :, :_L // 2], Xe, (((2,), (1,)), ((0,), (0,))),
        preferred_element_type=jnp.float32)
        + jax.lax.dot_general(
        attn[:, :, _L // 2:], Xo, (((2,), (1,)), ((0,), (0,))),
        preferred_element_type=jnp.float32))

    acc = jnp.zeros((bb, _DIN), jnp.float32)
    for h in range(_H):
        woh = Wo[:, h * _HD:(h + 1) * _HD]
        wvh = Wv[h * _HD:(h + 1) * _HD, :]
        Mh = jax.lax.dot_general(woh, wvh, (((1,), (0,)), ((), ())),
                                 preferred_element_type=jnp.float32)
        acc = acc + jax.lax.dot_general(mixed[:, h, :], Mh,
                                        (((1,), (1,)), ((), ())),
                                        preferred_element_type=jnp.float32)
    o_ref[...] = acc


def kernel(target_item, item_sequence, W_q, W_h, W_v, W_o):
    B = target_item.shape[0]
    grid = (B // _BB,)
    seq_packed = jnp.reshape(item_sequence.astype(jnp.bfloat16),
                             (B, _L // 2, 2 * _DIN))
    return pl.pallas_call(
        _body,
        grid=grid,
        in_specs=[
            pl.BlockSpec((_BB, _DIN), lambda i: (i, 0)),
            pl.BlockSpec((_BB, _L // 2, 2 * _DIN), lambda i: (i, 0, 0)),
            pl.BlockSpec((_DATT, _DIN), lambda i: (0, 0)),
            pl.BlockSpec((_DATT, _DIN), lambda i: (0, 0)),
            pl.BlockSpec((_DATT, _DIN), lambda i: (0, 0)),
            pl.BlockSpec((_DIN, _DATT), lambda i: (0, 0)),
        ],
        out_specs=pl.BlockSpec((_BB, _DIN), lambda i: (i, 0)),
        out_shape=jax.ShapeDtypeStruct((B, _DIN), jnp.float32),
    )(target_item, seq_packed, W_q, W_h, W_v, W_o)


# speedup vs baseline: 1.0599x; 1.0599x over previous
import jax
import jax.numpy as jnp
from jax.experimental import pallas as pl

_L = 200
_DIN = 64
_DATT = 128
_H = 4
_HD = 32
_K = 50
_BB = 128


def _limbs(a):
    a0 = a.astype(jnp.bfloat16)
    r1 = a - a0.astype(jnp.float32)
    a1 = r1.astype(jnp.bfloat16)
    r2 = r1 - a1.astype(jnp.float32)
    a2 = r2.astype(jnp.bfloat16)
    return a0, a1, a2


def _body(t_ref, x_ref, wq_ref, wh_ref, wv_ref, wo_ref, o_ref):
    tgt = t_ref[...]
    Xp = x_ref[...]
    Wq = wq_ref[...]
    Wh = wh_ref[...]
    Wv = wv_ref[...]
    Wo = wo_ref[...]
    bb = tgt.shape[0]

    Xbf = Xp.astype(jnp.bfloat16)
    Xe = Xbf[:, :, :_DIN]
    Xo = Xbf[:, :, _DIN:]
    qf = jax.lax.dot_general(tgt.astype(jnp.bfloat16),
                             Wq.astype(jnp.bfloat16),
                             (((1,), (1,)), ((), ())),
                             preferred_element_type=jnp.float32)
    Whbf = Wh.astype(jnp.bfloat16)
    rowh = jax.lax.broadcasted_iota(jnp.int32, (_DATT, _DIN), 0) // _HD
    Whbd = jnp.stack([jnp.where(rowh == h, Whbf, jnp.bfloat16(0))
                      for h in range(_H)], axis=1)
    qtil = jnp.zeros((bb, _H, _DIN), jnp.float32)
    for limb in _limbs(qf):
        qtil = qtil + jax.lax.dot_general(
            limb, Whbd, (((1,), (0,)), ((), ())),
            preferred_element_type=jnp.float32)
    se = jnp.zeros((bb, _H, _L // 2), jnp.float32)
    so = jnp.zeros((bb, _H, _L // 2), jnp.float32)
    for limb in _limbs(qtil):
        se = se + jax.lax.dot_general(
            limb, Xe, (((2,), (2,)), ((0,), (0,))),
            preferred_element_type=jnp.float32)
        so = so + jax.lax.dot_general(
            limb, Xo, (((2,), (2,)), ((0,), (0,))),
            preferred_element_type=jnp.float32)
    scores3 = jnp.concatenate([se, so], axis=-1)
    sT = jnp.swapaxes(scores3.reshape(bb * _H, _L), 0, 1) / (_HD ** 0.5)

    ibits = jax.lax.bitcast_convert_type(sT, jnp.int32)
    ikeys = jnp.where(ibits >= 0, ibits,
                      jnp.bitwise_xor(jnp.bitwise_not(ibits),
                                      jnp.int32(-2 ** 31)))

    t0 = jnp.full((1, bb * _H), jnp.int32(-2 ** 31), jnp.int32)

    def bs_body(i, t):
        inc = jax.lax.shift_left(jnp.int32(1), jnp.int32(31) - i)
        cand = t + inc
        cnt = jnp.sum((ikeys >= cand).astype(jnp.int32), axis=0,
                      keepdims=True)
        return jnp.where(cnt >= _K, cand, t)

    thr = jax.lax.fori_loop(0, 32, bs_body, t0)
    keep = ikeys >= thr

    m = jnp.max(sT, axis=0, keepdims=True)
    p = jnp.where(keep, jnp.exp(sT - m), 0.0)
    attnT = p / jnp.sum(p, axis=0, keepdims=True)
    attn = jnp.swapaxes(attnT.astype(jnp.bfloat16), 0, 1).reshape(bb, _H, _L)

    mixed = (jax.lax.dot_general(
        attn[:, :, :_L // 2], Xe, (((2,), (1,)), ((0,), (0,))),
        preferred_element_type=jnp.float32)
        + jax.lax.dot_general(
        attn[:, :, _L // 2:], Xo, (((2,), (1,)), ((0,), (0,))),
        preferred_element_type=jnp.float32))

    acc = jnp.zeros((bb, _DIN), jnp.float32)
    for h in range(_H):
        woh = Wo[:, h * _HD:(h + 1) * _HD]
        wvh = Wv[h * _HD:(h + 1) * _HD, :]
        Mh = jax.lax.dot_general(woh, wvh, (((1,), (0,)), ((), ())),
                                 preferred_element_type=jnp.float32)
        acc = acc + jax.lax.dot_general(mixed[:, h, :], Mh,
                                        (((1,), (1,)), ((), ())),
                                        preferred_element_type=jnp.float32)
    o_ref[...] = acc


def kernel(target_item, item_sequence, W_q, W_h, W_v, W_o):
    B = target_item.shape[0]
    grid = (B // _BB,)
    seq_packed = jnp.reshape(item_sequence, (B, _L // 2, 2 * _DIN))
    return pl.pallas_call(
        _body,
        grid=grid,
        in_specs=[
            pl.BlockSpec((_BB, _DIN), lambda i: (i, 0)),
            pl.BlockSpec((_BB, _L // 2, 2 * _DIN), lambda i: (i, 0, 0)),
            pl.BlockSpec((_DATT, _DIN), lambda i: (0, 0)),
            pl.BlockSpec((_DATT, _DIN), lambda i: (0, 0)),
            pl.BlockSpec((_DATT, _DIN), lambda i: (0, 0)),
            pl.BlockSpec((_DIN, _DATT), lambda i: (0, 0)),
        ],
        out_specs=pl.BlockSpec((_BB, _DIN), lambda i: (i, 0)),
        out_shape=jax.ShapeDtypeStruct((B, _DIN), jnp.float32),
    )(target_item, seq_packed, W_q, W_h, W_v, W_o)


# speedup vs baseline: 1.2734x; 1.2014x over previous
import jax
import jax.numpy as jnp
from jax.experimental import pallas as pl

_L = 200
_DIN = 64
_DATT = 128
_H = 4
_HD = 32
_K = 50
_BB = 128


def _limbs(a):
    a0 = a.astype(jnp.bfloat16)
    r1 = a - a0.astype(jnp.float32)
    a1 = r1.astype(jnp.bfloat16)
    r2 = r1 - a1.astype(jnp.float32)
    a2 = r2.astype(jnp.bfloat16)
    return a0, a1, a2


def _body(t_ref, x_ref, wq_ref, wh_ref, wv_ref, wo_ref, o_ref):
    tgt = t_ref[...]
    Xp = x_ref[...]
    Wq = wq_ref[...]
    Wh = wh_ref[...]
    Wv = wv_ref[...]
    Wo = wo_ref[...]
    bb = tgt.shape[0]

    Xbf = Xp.astype(jnp.bfloat16)
    Xe = Xbf[:, :, :_DIN]
    Xo = Xbf[:, :, _DIN:]
    qf = jax.lax.dot_general(tgt.astype(jnp.bfloat16),
                             Wq.astype(jnp.bfloat16),
                             (((1,), (1,)), ((), ())),
                             preferred_element_type=jnp.float32)
    Whbf = Wh.astype(jnp.bfloat16)
    rowh = jax.lax.broadcasted_iota(jnp.int32, (_DATT, _DIN), 0) // _HD
    Whbd = jnp.stack([jnp.where(rowh == h, Whbf, jnp.bfloat16(0))
                      for h in range(_H)], axis=1)
    qtil = jnp.zeros((bb, _H, _DIN), jnp.float32)
    for limb in _limbs(qf):
        qtil = qtil + jax.lax.dot_general(
            limb, Whbd, (((1,), (0,)), ((), ())),
            preferred_element_type=jnp.float32)
    se = jnp.zeros((bb, _H, _L // 2), jnp.float32)
    so = jnp.zeros((bb, _H, _L // 2), jnp.float32)
    for limb in _limbs(qtil)[:2]:
        se = se + jax.lax.dot_general(
            limb, Xe, (((2,), (2,)), ((0,), (0,))),
            preferred_element_type=jnp.float32)
        so = so + jax.lax.dot_general(
            limb, Xo, (((2,), (2,)), ((0,), (0,))),
            preferred_element_type=jnp.float32)
    scores3 = jnp.concatenate([se, so], axis=-1)
    sT = jnp.swapaxes(scores3.reshape(bb * _H, _L), 0, 1) / (_HD ** 0.5)

    ibits = jax.lax.bitcast_convert_type(sT, jnp.int32)
    ikeys = jnp.where(ibits >= 0, ibits,
                      jnp.bitwise_xor(jnp.bitwise_not(ibits),
                                      jnp.int32(-2 ** 31)))

    t0 = jnp.full((1, bb * _H), jnp.int32(-2 ** 31), jnp.int32)

    def bs_body(i, t):
        inc = jax.lax.shift_left(jnp.int32(1), jnp.int32(31) - i)
        cand = t + inc
        cnt = jnp.sum((ikeys >= cand).astype(jnp.int32), axis=0,
                      keepdims=True)
        return jnp.where(cnt >= _K, cand, t)

    thr = jax.lax.fori_loop(0, 32, bs_body, t0, unroll=8)
    keep = ikeys >= thr

    m = jnp.max(sT, axis=0, keepdims=True)
    p = jnp.where(keep, jnp.exp(sT - m), 0.0)
    attnT = p / jnp.sum(p, axis=0, keepdims=True)
    attn = jnp.swapaxes(attnT.astype(jnp.bfloat16), 0, 1).reshape(bb, _H, _L)

    mixed = (jax.lax.dot_general(
        attn[:, :, :_L // 2], Xe, (((2,), (1,)), ((0,), (0,))),
        preferred_element_type=jnp.float32)
        + jax.lax.dot_general(
        attn[:, :, _L // 2:], Xo, (((2,), (1,)), ((0,), (0,))),
        preferred_element_type=jnp.float32))

    acc = jnp.zeros((bb, _DIN), jnp.float32)
    for h in range(_H):
        woh = Wo[:, h * _HD:(h + 1) * _HD]
        wvh = Wv[h * _HD:(h + 1) * _HD, :]
        Mh = jax.lax.dot_general(woh, wvh, (((1,), (0,)), ((), ())),
                                 preferred_element_type=jnp.float32)
        acc = acc + jax.lax.dot_general(mixed[:, h, :], Mh,
                                        (((1,), (1,)), ((), ())),
                                        preferred_element_type=jnp.float32)
    o_ref[...] = acc


def kernel(target_item, item_sequence, W_q, W_h, W_v, W_o):
    B = target_item.shape[0]
    grid = (B // _BB,)
    seq_packed = jnp.reshape(item_sequence, (B, _L // 2, 2 * _DIN))
    return pl.pallas_call(
        _body,
        grid=grid,
        in_specs=[
            pl.BlockSpec((_BB, _DIN), lambda i: (i, 0)),
            pl.BlockSpec((_BB, _L // 2, 2 * _DIN), lambda i: (i, 0, 0)),
            pl.BlockSpec((_DATT, _DIN), lambda i: (0, 0)),
            pl.BlockSpec((_DATT, _DIN), lambda i: (0, 0)),
            pl.BlockSpec((_DATT, _DIN), lambda i: (0, 0)),
            pl.BlockSpec((_DIN, _DATT), lambda i: (0, 0)),
        ],
        out_specs=pl.BlockSpec((_BB, _DIN), lambda i: (i, 0)),
        out_shape=jax.ShapeDtypeStruct((B, _DIN), jnp.float32),
    )(target_item, seq_packed, W_q, W_h, W_v, W_o)


# speedup vs baseline: 1.3066x; 1.0260x over previous
import jax
import jax.numpy as jnp
from jax.experimental import pallas as pl

_L = 200
_DIN = 64
_DATT = 128
_H = 4
_HD = 32
_K = 50
_BB = 128


def _limbs(a):
    a0 = a.astype(jnp.bfloat16)
    r1 = a - a0.astype(jnp.float32)
    a1 = r1.astype(jnp.bfloat16)
    r2 = r1 - a1.astype(jnp.float32)
    a2 = r2.astype(jnp.bfloat16)
    return a0, a1, a2


def _body(t_ref, x_ref, wq_ref, wh_ref, wv_ref, wo_ref, o_ref):
    tgt = t_ref[...]
    Xp = x_ref[...]
    Wq = wq_ref[...]
    Wh = wh_ref[...]
    Wv = wv_ref[...]
    Wo = wo_ref[...]
    bb = tgt.shape[0]

    Xbf = Xp.astype(jnp.bfloat16)
    Xe = Xbf[:, :, :_DIN]
    Xo = Xbf[:, :, _DIN:]
    qf = jax.lax.dot_general(tgt.astype(jnp.bfloat16),
                             Wq.astype(jnp.bfloat16),
                             (((1,), (1,)), ((), ())),
                             preferred_element_type=jnp.float32)
    Whbf = Wh.astype(jnp.bfloat16)
    rowh = jax.lax.broadcasted_iota(jnp.int32, (_DATT, _DIN), 0) // _HD
    Whbd = jnp.stack([jnp.where(rowh == h, Whbf, jnp.bfloat16(0))
                      for h in range(_H)], axis=1)
    qtil = jnp.zeros((bb, _H, _DIN), jnp.float32)
    for limb in _limbs(qf):
        qtil = qtil + jax.lax.dot_general(
            limb, Whbd, (((1,), (0,)), ((), ())),
            preferred_element_type=jnp.float32)
    se = jnp.zeros((bb, _H, _L // 2), jnp.float32)
    so = jnp.zeros((bb, _H, _L // 2), jnp.float32)
    for limb in _limbs(qtil)[:2]:
        se = se + jax.lax.dot_general(
            limb, Xe, (((2,), (2,)), ((0,), (0,))),
            preferred_element_type=jnp.float32)
        so = so + jax.lax.dot_general(
            limb, Xo, (((2,), (2,)), ((0,), (0,))),
            preferred_element_type=jnp.float32)
    scores3 = jnp.concatenate([se, so], axis=-1)
    sT = jnp.swapaxes(scores3.reshape(bb * _H, _L), 0, 1) / (_HD ** 0.5)

    ibits = jax.lax.bitcast_convert_type(sT, jnp.int32)
    ikeys = jnp.where(ibits >= 0, ibits,
                      jnp.bitwise_xor(jnp.bitwise_not(ibits),
                                      jnp.int32(-2 ** 31)))

    t0 = jnp.full((1, bb * _H), jnp.int32(-2 ** 31), jnp.int32)

    def bs_body(i, t):
        inc = jax.lax.shift_left(jnp.int32(1), jnp.int32(31) - i)
        cand = t + inc
        cnt = jnp.sum((ikeys >= cand).astype(jnp.int32), axis=0,
                      keepdims=True)
        return jnp.where(cnt >= _K, cand, t)

    thr = jax.lax.fori_loop(0, 32, bs_body, t0, unroll=32)
    keep = ikeys >= thr

    m = jnp.max(sT, axis=0, keepdims=True)
    p = jnp.where(keep, jnp.exp(sT - m), 0.0)
    attnT = p / jnp.sum(p, axis=0, keepdims=True)
    attn = jnp.swapaxes(attnT.astype(jnp.bfloat16), 0, 1).reshape(bb, _H, _L)

    mixed = (jax.lax.dot_general(
        attn[:, :, :_L // 2], Xe, (((2,), (1,)), ((0,), (0,))),
        preferred_element_type=jnp.float32)
        + jax.lax.dot_general(
        attn[:, :, _L // 2:], Xo, (((2,), (1,)), ((0,), (0,))),
        preferred_element_type=jnp.float32))

    acc = jnp.zeros((bb, _DIN), jnp.float32)
    for h in range(_H):
        woh = Wo[:, h * _HD:(h + 1) * _HD]
        wvh = Wv[h * _HD:(h + 1) * _HD, :]
        Mh = jax.lax.dot_general(woh, wvh, (((1,), (0,)), ((), ())),
                                 preferred_element_type=jnp.float32)
        acc = acc + jax.lax.dot_general(mixed[:, h, :], Mh,
                                        (((1,), (1,)), ((), ())),
                                        preferred_element_type=jnp.float32)
    o_ref[...] = acc


def kernel(target_item, item_sequence, W_q, W_h, W_v, W_o):
    B = target_item.shape[0]
    grid = (B // _BB,)
    seq_packed = jnp.reshape(item_sequence, (B, _L // 2, 2 * _DIN))
    return pl.pallas_call(
        _body,
        grid=grid,
        in_specs=[
            pl.BlockSpec((_BB, _DIN), lambda i: (i, 0)),
            pl.BlockSpec((_BB, _L // 2, 2 * _DIN), lambda i: (i, 0, 0)),
            pl.BlockSpec((_DATT, _DIN), lambda i: (0, 0)),
            pl.BlockSpec((_DATT, _DIN), lambda i: (0, 0)),
            pl.BlockSpec((_DATT, _DIN), lambda i: (0, 0)),
            pl.BlockSpec((_DIN, _DATT), lambda i: (0, 0)),
        ],
        out_specs=pl.BlockSpec((_BB, _DIN), lambda i: (i, 0)),
        out_shape=jax.ShapeDtypeStruct((B, _DIN), jnp.float32),
    )(target_item, seq_packed, W_q, W_h, W_v, W_o)


# speedup vs baseline: 1.3793x; 1.0557x over previous
import jax
import jax.numpy as jnp
from jax.experimental import pallas as pl
from jax.experimental.pallas import tpu as pltpu

_L = 200
_DIN = 64
_DATT = 128
_H = 4
_HD = 32
_K = 50
_BB = 256


def _limbs(a):
    a0 = a.astype(jnp.bfloat16)
    r1 = a - a0.astype(jnp.float32)
    a1 = r1.astype(jnp.bfloat16)
    r2 = r1 - a1.astype(jnp.float32)
    a2 = r2.astype(jnp.bfloat16)
    return a0, a1, a2


def _body(t_ref, x_ref, wq_ref, wh_ref, wv_ref, wo_ref, o_ref):
    tgt = t_ref[...]
    Xp = x_ref[...]
    Wq = wq_ref[...]
    Wh = wh_ref[...]
    Wv = wv_ref[...]
    Wo = wo_ref[...]
    bb = tgt.shape[0]

    Xbf = Xp.astype(jnp.bfloat16)
    Xe = Xbf[:, :, :_DIN]
    Xo = Xbf[:, :, _DIN:]
    qf = jax.lax.dot_general(tgt.astype(jnp.bfloat16),
                             Wq.astype(jnp.bfloat16),
                             (((1,), (1,)), ((), ())),
                             preferred_element_type=jnp.float32)
    Whbf = Wh.astype(jnp.bfloat16)
    rowh = jax.lax.broadcasted_iota(jnp.int32, (_DATT, _DIN), 0) // _HD
    Whbd = jnp.stack([jnp.where(rowh == h, Whbf, jnp.bfloat16(0))
                      for h in range(_H)], axis=1)
    qtil = jnp.zeros((bb, _H, _DIN), jnp.float32)
    for limb in _limbs(qf):
        qtil = qtil + jax.lax.dot_general(
            limb, Whbd, (((1,), (0,)), ((), ())),
            preferred_element_type=jnp.float32)
    se = jnp.zeros((bb, _H, _L // 2), jnp.float32)
    so = jnp.zeros((bb, _H, _L // 2), jnp.float32)
    for limb in _limbs(qtil)[:2]:
        se = se + jax.lax.dot_general(
            limb, Xe, (((2,), (2,)), ((0,), (0,))),
            preferred_element_type=jnp.float32)
        so = so + jax.lax.dot_general(
            limb, Xo, (((2,), (2,)), ((0,), (0,))),
            preferred_element_type=jnp.float32)
    scores3 = jnp.concatenate([se, so], axis=-1)
    sT = jnp.swapaxes(scores3.reshape(bb * _H, _L), 0, 1) / (_HD ** 0.5)

    ibits = jax.lax.bitcast_convert_type(sT, jnp.int32)
    ikeys = jnp.where(ibits >= 0, ibits,
                      jnp.bitwise_xor(jnp.bitwise_not(ibits),
                                      jnp.int32(-2 ** 31)))

    t0 = jnp.full((1, bb * _H), jnp.int32(-2 ** 31), jnp.int32)

    def bs_body(i, t):
        inc = jax.lax.shift_left(jnp.int32(1), jnp.int32(31) - i)
        cand = t + inc
        cnt = jnp.sum((ikeys >= cand).astype(jnp.int32), axis=0,
                      keepdims=True)
        return jnp.where(cnt >= _K, cand, t)

    thr = jax.lax.fori_loop(0, 32, bs_body, t0, unroll=32)
    keep = ikeys >= thr

    m = jnp.max(sT, axis=0, keepdims=True)
    p = jnp.where(keep, jnp.exp(sT - m), 0.0)
    attnT = p / jnp.sum(p, axis=0, keepdims=True)
    attn = jnp.swapaxes(attnT.astype(jnp.bfloat16), 0, 1).reshape(bb, _H, _L)

    mixed = (jax.lax.dot_general(
        attn[:, :, :_L // 2], Xe, (((2,), (1,)), ((0,), (0,))),
        preferred_element_type=jnp.float32)
        + jax.lax.dot_general(
        attn[:, :, _L // 2:], Xo, (((2,), (1,)), ((0,), (0,))),
        preferred_element_type=jnp.float32))

    acc = jnp.zeros((bb, _DIN), jnp.float32)
    for h in range(_H):
        woh = Wo[:, h * _HD:(h + 1) * _HD]
        wvh = Wv[h * _HD:(h + 1) * _HD, :]
        Mh = jax.lax.dot_general(woh, wvh, (((1,), (0,)), ((), ())),
                                 preferred_element_type=jnp.float32)
        acc = acc + jax.lax.dot_general(mixed[:, h, :], Mh,
                                        (((1,), (1,)), ((), ())),
                                        preferred_element_type=jnp.float32)
    o_ref[...] = acc


def kernel(target_item, item_sequence, W_q, W_h, W_v, W_o):
    B = target_item.shape[0]
    grid = (B // _BB,)
    seq_packed = jnp.reshape(item_sequence, (B, _L // 2, 2 * _DIN))
    return pl.pallas_call(
        _body,
        grid=grid,
        in_specs=[
            pl.BlockSpec((_BB, _DIN), lambda i: (i, 0)),
            pl.BlockSpec((_BB, _L // 2, 2 * _DIN), lambda i: (i, 0, 0)),
            pl.BlockSpec((_DATT, _DIN), lambda i: (0, 0)),
            pl.BlockSpec((_DATT, _DIN), lambda i: (0, 0)),
            pl.BlockSpec((_DATT, _DIN), lambda i: (0, 0)),
            pl.BlockSpec((_DIN, _DATT), lambda i: (0, 0)),
        ],
        out_specs=pl.BlockSpec((_BB, _DIN), lambda i: (i, 0)),
        out_shape=jax.ShapeDtypeStruct((B, _DIN), jnp.float32),
        compiler_params=pltpu.CompilerParams(
            vmem_limit_bytes=64 * 1024 * 1024),
    )(target_item, seq_packed, W_q, W_h, W_v, W_o)
